# baseline (speedup 1.0000x reference)
import functools

import jax
import jax.numpy as jnp
from jax.experimental import pallas as pl
from jax.experimental.pallas import tpu as pltpu

_EPS = 1e-06
_CAPACITY_FACTOR = 1.0


def _gate_kernel(x_ref, w_ref, b_ref, out_ref, lg_ref, denom_ref, *,
                 block_tokens, num_blocks, capacity):
    i = pl.program_id(0)

    @pl.when(i == 0)
    def _init():
        denom_ref[:] = jnp.zeros_like(denom_ref)

    j = jnp.maximum(i - 1, 0)
    logits = lg_ref[jax.lax.rem(i + 1, 2)]

    m = jnp.max(logits, axis=-1, keepdims=True)
    idx = jnp.argmax(logits, axis=-1)[:, None]

    s = jnp.sum(jnp.exp(logits - m), axis=-1, keepdims=True)
    lanes = jax.lax.broadcasted_iota(jnp.int32, logits.shape, 1)
    masked = jnp.where(lanes == idx, 1.0 / s, 0.0)

    out_ref[pl.ds(j * block_tokens, block_tokens), :] = masked
    denom_ref[:] += jnp.where(
        i > 0, jnp.sum(masked, axis=0, keepdims=True), 0.0)

    lg_ref[jax.lax.rem(i, 2)] = jax.lax.dot_general(
        x_ref[:], w_ref[:],
        dimension_numbers=(((1,), (1,)), ((), ())),
        preferred_element_type=jnp.float32,
    ) + b_ref[:]

    @pl.when(i == num_blocks)
    def _finalize():
        out_ref[:] = out_ref[:] / (denom_ref[:] + _EPS) * capacity


def kernel(x, W, b):
    tokens, dim = x.shape
    num_experts = W.shape[0]
    capacity = int(_CAPACITY_FACTOR * tokens)

    block_tokens = 1024
    num_blocks = tokens // block_tokens

    body = functools.partial(
        _gate_kernel,
        block_tokens=block_tokens,
        num_blocks=num_blocks,
        capacity=float(capacity),
    )

    return pl.pallas_call(
        body,
        grid=(num_blocks + 1,),
        in_specs=[
            pl.BlockSpec((block_tokens, dim),
                         lambda i, nb=num_blocks - 1: (jnp.minimum(i, nb), 0)),
            pl.BlockSpec((num_experts, dim), lambda i: (0, 0)),
            pl.BlockSpec((1, num_experts), lambda i: (0, 0)),
        ],
        out_specs=pl.BlockSpec((tokens, num_experts), lambda i: (0, 0)),
        out_shape=jax.ShapeDtypeStruct((tokens, num_experts), jnp.float32),
        scratch_shapes=[
            pltpu.VMEM((2, block_tokens, num_experts), jnp.float32),
            pltpu.VMEM((1, num_experts), jnp.float32),
        ],
    )(x, W, b.reshape(1, num_experts))
